# Initial kernel scaffold; baseline (speedup 1.0000x reference)
import jax
import jax.numpy as jnp
from jax.experimental import pallas as pl


def kernel(x_user, x_item, edge_index_u2i, edge_index_i2u, edge_label_index, W1_l_u2i, b1_u2i, W1_r_u2i, W1_l_i2u, b1_i2u, W1_r_i2u, W2_l_u2i, b2_u2i, W2_r_u2i, W2_l_i2u, b2_i2u, W2_r_i2u):
    raise NotImplementedError("write your pallas kernel here")


# speedup vs baseline: 4.6025x; 4.6025x over previous
import functools

import jax
import jax.numpy as jnp
from jax import lax
from jax.experimental import pallas as pl
from jax.experimental.pallas import tpu as pltpu
from jax.experimental.pallas import tpu_sc as plsc

N_USER = 10000
N_ITEM = 10000
E = 320000
E_LBL = 100000
F_IN = 128
F_HID = 128
F_OUT = 64

NC = 2
NS = 16

EDGE_CHUNK = 80
EDGES_PER_TILE = E // NS
CHUNKS_PER_TILE = EDGES_PER_TILE // EDGE_CHUNK
N_CHUNK_ROWS = E // EDGE_CHUNK

STRIPE = 624
TAIL_BASE = STRIPE * NS
TAIL = N_USER - TAIL_BASE

LBL_PAD = 102400
LBL_CHUNK = 128
LBL_CHUNKS_PER_TILE = LBL_PAD // (NC * NS) // LBL_CHUNK

_MESH = plsc.VectorSubcoreMesh(core_axis_name="c", subcore_axis_name="s")


def _stripe_copy(s, src, dst):
  r0 = s * STRIPE
  pltpu.sync_copy(src.at[pl.ds(r0, STRIPE)], dst.at[pl.ds(r0, STRIPE)])

  @pl.when(s == NS - 1)
  def _():
    pltpu.sync_copy(src.at[pl.ds(TAIL_BASE, TAIL)],
                    dst.at[pl.ds(TAIL_BASE, TAIL)])


@functools.partial(
    pl.kernel,
    out_type=[
        jax.ShapeDtypeStruct((NC, N_USER, F_OUT), jnp.float32),
        jax.ShapeDtypeStruct((NC, N_USER, F_OUT), jnp.float32),
        jax.ShapeDtypeStruct((N_USER, 8), jnp.float32),
        jax.ShapeDtypeStruct((N_USER, 8), jnp.float32),
    ],
    mesh=_MESH,
    compiler_params=pltpu.CompilerParams(use_tc_tiling_on_sc=False),
    scratch_types=[
        pltpu.VMEM((CHUNKS_PER_TILE, EDGE_CHUNK), jnp.int32),
        pltpu.VMEM((CHUNKS_PER_TILE, EDGE_CHUNK), jnp.int32),
        pltpu.VMEM((EDGE_CHUNK, F_OUT), jnp.float32),
        pltpu.VMEM((EDGE_CHUNK, 8), jnp.float32),
        pltpu.VMEM_SHARED((N_USER, F_OUT), jnp.float32),
        pltpu.VMEM_SHARED((N_USER, 8), jnp.float32),
        pltpu.SemaphoreType.DMA,
    ],
)
def _seg_sum_l1(x_user_h, x_item_h, src_ui, dst_ui, src_iu, dst_iu,
                zeros_64, zeros_8, ones_8,
                agg_item_h, agg_user_h, cnt_item, cnt_user,
                idx_s, idx_d, rows, ones_v, acc, cntacc, sem):
  c = lax.axis_index("c")
  s = lax.axis_index("s")

  pltpu.sync_copy(ones_8, ones_v)

  def phase(x_h, src_r, dst_r, agg_out, cnt_out, cnt_core):
    _stripe_copy(s, zeros_64, acc)
    _stripe_copy(s, zeros_8, cntacc)
    plsc.subcore_barrier()

    def run(x_view, do_cnt):
      pltpu.sync_copy(src_r.at[s], idx_s)
      pltpu.sync_copy(dst_r.at[s], idx_d)

      def chunk(j, carry):
        pltpu.async_copy(x_view.at[idx_s.at[j]], rows, sem).wait()
        pltpu.sync_copy(rows, acc.at[idx_d.at[j]], add=True)
        if do_cnt:
          pltpu.sync_copy(ones_v, cntacc.at[idx_d.at[j]], add=True)
        return carry

      lax.fori_loop(0, CHUNKS_PER_TILE, chunk, 0)

    @pl.when(c == 0)
    def _():
      run(x_h.at[0], cnt_core == 0)

    @pl.when(c != 0)
    def _():
      run(x_h.at[1], cnt_core == 1)

    plsc.subcore_barrier()
    _stripe_copy(s, acc, agg_out.at[c])

    @pl.when(c == cnt_core)
    def _():
      _stripe_copy(s, cntacc, cnt_out)

  phase(x_user_h, src_ui, dst_ui, agg_item_h, cnt_item, 0)
  plsc.subcore_barrier()
  phase(x_item_h, src_iu, dst_iu, agg_user_h, cnt_user, 1)


@functools.partial(
    pl.kernel,
    out_type=[jax.ShapeDtypeStruct((N_USER, F_OUT), jnp.float32)
              for _ in range(2)],
    mesh=_MESH,
    compiler_params=pltpu.CompilerParams(use_tc_tiling_on_sc=False),
    scratch_types=[
        pltpu.VMEM((CHUNKS_PER_TILE, EDGE_CHUNK), jnp.int32),
        pltpu.VMEM((CHUNKS_PER_TILE, EDGE_CHUNK), jnp.int32),
        pltpu.VMEM((EDGE_CHUNK, F_OUT), jnp.float32),
        pltpu.VMEM_SHARED((N_USER, F_OUT), jnp.float32),
        pltpu.SemaphoreType.DMA,
    ],
)
def _seg_sum_l2(q_u, q_i, src_ui, dst_ui, src_iu, dst_iu, zeros_64,
                agg2_item, agg2_user,
                idx_s, idx_d, rows, acc, sem):
  c = lax.axis_index("c")
  s = lax.axis_index("s")

  _stripe_copy(s, zeros_64, acc)
  plsc.subcore_barrier()

  def process(x_hbm, src_r, dst_r):
    pltpu.sync_copy(src_r.at[s], idx_s)
    pltpu.sync_copy(dst_r.at[s], idx_d)

    def chunk(j, carry):
      pltpu.async_copy(x_hbm.at[idx_s.at[j]], rows, sem).wait()
      pltpu.sync_copy(rows, acc.at[idx_d.at[j]], add=True)
      return carry

    lax.fori_loop(0, CHUNKS_PER_TILE, chunk, 0)

  @pl.when(c == 0)
  def _():
    process(q_u, src_ui, dst_ui)

  @pl.when(c != 0)
  def _():
    process(q_i, src_iu, dst_iu)

  plsc.subcore_barrier()

  @pl.when(c == 0)
  def _():
    _stripe_copy(s, acc, agg2_item)

  @pl.when(c != 0)
  def _():
    _stripe_copy(s, acc, agg2_user)


@functools.partial(
    pl.kernel,
    out_type=[jax.ShapeDtypeStruct((LBL_PAD, F_OUT), jnp.float32)
              for _ in range(2)],
    mesh=_MESH,
    compiler_params=pltpu.CompilerParams(use_tc_tiling_on_sc=False),
    scratch_types=[
        pltpu.VMEM((LBL_CHUNKS_PER_TILE, LBL_CHUNK), jnp.int32),
        pltpu.VMEM((LBL_CHUNKS_PER_TILE, LBL_CHUNK), jnp.int32),
        pltpu.VMEM((LBL_CHUNK, F_OUT), jnp.float32),
        pltpu.VMEM((LBL_CHUNK, F_OUT), jnp.float32),
        pltpu.SemaphoreType.DMA,
        pltpu.SemaphoreType.DMA,
    ],
)
def _pair_gather(z_user, z_item, el0_r, el1_r, f1_out, f2_out,
                 idx0, idx1, rows0, rows1, sem0, sem1):
  c = lax.axis_index("c")
  s = lax.axis_index("s")
  w = s * NC + c
  base = w * LBL_CHUNKS_PER_TILE
  pltpu.sync_copy(el0_r.at[w], idx0)
  pltpu.sync_copy(el1_r.at[w], idx1)

  def chunk(j, carry):
    out_base = (base + j) * LBL_CHUNK
    cp0 = pltpu.async_copy(z_user.at[idx0.at[j]], rows0, sem0)
    cp1 = pltpu.async_copy(z_item.at[idx1.at[j]], rows1, sem1)
    cp0.wait()
    pltpu.sync_copy(rows0, f1_out.at[pl.ds(out_base, LBL_CHUNK)])
    cp1.wait()
    pltpu.sync_copy(rows1, f2_out.at[pl.ds(out_base, LBL_CHUNK)])
    return carry

  lax.fori_loop(0, LBL_CHUNKS_PER_TILE, chunk, 0)


def _dotT(x, w):
  return lax.dot_general(x, w, (((1,), (1,)), ((), ())),
                         preferred_element_type=jnp.float32)


_ROWS_BLK = 1000
_GRID = N_USER // _ROWS_BLK


def _blk(width):
  return pl.BlockSpec((_ROWS_BLK, width), lambda i: (i, 0))


def _full(shape):
  return pl.BlockSpec(shape, lambda i: tuple(0 for _ in shape))


def _hidden_tc_body(agg_i_lo, agg_i_hi, cnt_i_ref, x_i_ref,
                    agg_u_lo, agg_u_hi, cnt_u_ref, x_u_ref,
                    w1l_ui, w1r_ui, b1_ui, w1l_iu, w1r_iu, b1_iu,
                    w2l_ui, w2l_iu, w2r_ui, b2_ui, w2r_iu, b2_iu,
                    q_u_ref, q_i_ref, r2_i_ref, r2_u_ref):
  inv_i = 1.0 / jnp.maximum(cnt_i_ref[:, 0:1], 1.0)
  inv_u = 1.0 / jnp.maximum(cnt_u_ref[:, 0:1], 1.0)
  agg_i = jnp.concatenate([agg_i_lo[...], agg_i_hi[...]], axis=1)
  agg_u = jnp.concatenate([agg_u_lo[...], agg_u_hi[...]], axis=1)
  h_item = jax.nn.relu(_dotT(agg_i * inv_i, w1l_ui[...]) +
                       _dotT(x_i_ref[...], w1r_ui[...]) + b1_ui[...])
  h_user = jax.nn.relu(_dotT(agg_u * inv_u, w1l_iu[...]) +
                       _dotT(x_u_ref[...], w1r_iu[...]) + b1_iu[...])
  q_u_ref[...] = _dotT(h_user, w2l_ui[...])
  q_i_ref[...] = _dotT(h_item, w2l_iu[...])
  r2_i_ref[...] = _dotT(h_item, w2r_ui[...]) + b2_ui[...]
  r2_u_ref[...] = _dotT(h_user, w2r_iu[...]) + b2_iu[...]


_hidden_tc = pl.pallas_call(
    _hidden_tc_body,
    grid=(_GRID,),
    in_specs=[_blk(F_OUT), _blk(F_OUT), _blk(8), _blk(F_IN),
              _blk(F_OUT), _blk(F_OUT), _blk(8), _blk(F_IN),
              _full((F_HID, F_IN)), _full((F_HID, F_IN)), _full((1, F_HID)),
              _full((F_HID, F_IN)), _full((F_HID, F_IN)), _full((1, F_HID)),
              _full((F_OUT, F_HID)), _full((F_OUT, F_HID)),
              _full((F_OUT, F_HID)), _full((1, F_OUT)),
              _full((F_OUT, F_HID)), _full((1, F_OUT))],
    out_specs=[_blk(F_OUT)] * 4,
    out_shape=[jax.ShapeDtypeStruct((N_USER, F_OUT), jnp.float32)] * 4,
)


def _final_tc_body(agg2_i_ref, cnt_i_ref, r2_i_ref, agg2_u_ref, cnt_u_ref,
                   r2_u_ref, z_i_ref, z_u_ref):
  inv_i = 1.0 / jnp.maximum(cnt_i_ref[:, 0:1], 1.0)
  inv_u = 1.0 / jnp.maximum(cnt_u_ref[:, 0:1], 1.0)
  z_i_ref[...] = agg2_i_ref[...] * inv_i + r2_i_ref[...]
  z_u_ref[...] = agg2_u_ref[...] * inv_u + r2_u_ref[...]


_final_tc = pl.pallas_call(
    _final_tc_body,
    grid=(_GRID,),
    in_specs=[_blk(F_OUT), _blk(8), _blk(F_OUT)] * 2,
    out_specs=[_blk(F_OUT)] * 2,
    out_shape=[jax.ShapeDtypeStruct((N_USER, F_OUT), jnp.float32)] * 2,
)


_DOT_BLK = 1024


def _pair_dot_body(f1_ref, f2_ref, out_ref):
  out_ref[...] = jnp.sum(f1_ref[...] * f2_ref[...], axis=1)


_pair_dot_tc = pl.pallas_call(
    _pair_dot_body,
    grid=(LBL_PAD // _DOT_BLK,),
    in_specs=[pl.BlockSpec((_DOT_BLK, F_OUT), lambda i: (i, 0))] * 2,
    out_specs=pl.BlockSpec((_DOT_BLK,), lambda i: (i,)),
    out_shape=jax.ShapeDtypeStruct((LBL_PAD,), jnp.float32),
)


@jax.jit
def kernel(x_user, x_item, edge_index_u2i, edge_index_i2u, edge_label_index,
           W1_l_u2i, b1_u2i, W1_r_u2i, W1_l_i2u, b1_i2u, W1_r_i2u,
           W2_l_u2i, b2_u2i, W2_r_u2i, W2_l_i2u, b2_i2u, W2_r_i2u):
  eshape = (NS, CHUNKS_PER_TILE, EDGE_CHUNK)
  src_ui = edge_index_u2i[0].astype(jnp.int32).reshape(eshape)
  dst_ui = edge_index_u2i[1].astype(jnp.int32).reshape(eshape)
  src_iu = edge_index_i2u[0].astype(jnp.int32).reshape(eshape)
  dst_iu = edge_index_i2u[1].astype(jnp.int32).reshape(eshape)

  zeros_64 = jnp.zeros((N_USER, F_OUT), jnp.float32)
  zeros_8 = jnp.zeros((N_USER, 8), jnp.float32)
  ones_8 = jnp.ones((EDGE_CHUNK, 8), jnp.float32)

  x_user_h = x_user.reshape(N_USER, NC, F_OUT).transpose(1, 0, 2)
  x_item_h = x_item.reshape(N_ITEM, NC, F_OUT).transpose(1, 0, 2)

  agg_item_h, agg_user_h, cnt_item, cnt_user = _seg_sum_l1(
      x_user_h, x_item_h, src_ui, dst_ui, src_iu, dst_iu,
      zeros_64, zeros_8, ones_8)

  q_u, q_i, r2_item, r2_user = _hidden_tc(
      agg_item_h[0], agg_item_h[1], cnt_item, x_item,
      agg_user_h[0], agg_user_h[1], cnt_user, x_user,
      W1_l_u2i, W1_r_u2i, b1_u2i.reshape(1, F_HID),
      W1_l_i2u, W1_r_i2u, b1_i2u.reshape(1, F_HID),
      W2_l_u2i, W2_l_i2u, W2_r_u2i, b2_u2i.reshape(1, F_OUT),
      W2_r_i2u, b2_i2u.reshape(1, F_OUT))

  agg2_item, agg2_user = _seg_sum_l2(
      q_u, q_i, src_ui, dst_ui, src_iu, dst_iu, zeros_64)

  z_item, z_user = _final_tc(agg2_item, cnt_item, r2_item,
                             agg2_user, cnt_user, r2_user)

  el = edge_label_index.astype(jnp.int32)
  pad = LBL_PAD - E_LBL
  lshape = (NC * NS, LBL_CHUNKS_PER_TILE, LBL_CHUNK)
  el0_r = jnp.concatenate([el[0], jnp.zeros((pad,), jnp.int32)]).reshape(lshape)
  el1_r = jnp.concatenate([el[1], jnp.zeros((pad,), jnp.int32)]).reshape(lshape)

  f1, f2 = _pair_gather(z_user, z_item, el0_r, el1_r)
  pred = _pair_dot_tc(f1, f2)
  return pred[:E_LBL]


# speedup vs baseline: 6.5508x; 1.4233x over previous
import functools

import jax
import jax.numpy as jnp
from jax import lax
from jax.experimental import pallas as pl
from jax.experimental.pallas import tpu as pltpu
from jax.experimental.pallas import tpu_sc as plsc

N_USER = 10000
N_ITEM = 10000
E = 320000
E_LBL = 100000
F_IN = 128
F_HID = 128
F_OUT = 64

NC = 2
NS = 16

EDGE_CHUNK = 80
EDGES_PER_TILE = E // NS
CHUNKS_PER_TILE = EDGES_PER_TILE // EDGE_CHUNK
CHUNK_PAIRS = CHUNKS_PER_TILE // 2
N_CHUNK_ROWS = E // EDGE_CHUNK

STRIPE = 624
TAIL_BASE = STRIPE * NS
TAIL = N_USER - TAIL_BASE

LBL_PAD = 102400
LBL_CHUNK = 128
LBL_CHUNKS_PER_TILE = LBL_PAD // (NC * NS) // LBL_CHUNK

_MESH = plsc.VectorSubcoreMesh(core_axis_name="c", subcore_axis_name="s")


def _edge_loop(x_view, idx_s, idx_d, rows0, rows1, sem0, sem1, drain_hbm,
               acc, cnt, ones_v, do_cnt):
  drain = drain_hbm.at[pl.ds(0, EDGE_CHUNK)]

  def scatter(rows, j):
    pltpu.sync_copy(rows, acc.at[idx_d.at[j]], add=True)
    if do_cnt:
      pltpu.sync_copy(ones_v, cnt.at[idx_d.at[j]], add=True)

  pltpu.async_copy(x_view.at[idx_s.at[0]], rows0, sem0)

  def pair(i, carry):
    j0 = 2 * i
    pltpu.make_async_copy(drain, rows0, sem0).wait()
    pltpu.async_copy(x_view.at[idx_s.at[j0 + 1]], rows1, sem1)
    scatter(rows0, j0)

    @pl.when(i < CHUNK_PAIRS - 1)
    def _():
      pltpu.async_copy(x_view.at[idx_s.at[j0 + 2]], rows0, sem0)

    pltpu.make_async_copy(drain, rows1, sem1).wait()
    scatter(rows1, j0 + 1)
    return carry

  lax.fori_loop(0, CHUNK_PAIRS, pair, 0)


def _stripe_copy(s, src, dst):
  r0 = s * STRIPE
  pltpu.sync_copy(src.at[pl.ds(r0, STRIPE)], dst.at[pl.ds(r0, STRIPE)])

  @pl.when(s == NS - 1)
  def _():
    pltpu.sync_copy(src.at[pl.ds(TAIL_BASE, TAIL)],
                    dst.at[pl.ds(TAIL_BASE, TAIL)])


@functools.partial(
    pl.kernel,
    out_type=[
        jax.ShapeDtypeStruct((NC, N_USER, F_OUT), jnp.float32),
        jax.ShapeDtypeStruct((NC, N_USER, F_OUT), jnp.float32),
        jax.ShapeDtypeStruct((N_USER, 8), jnp.float32),
        jax.ShapeDtypeStruct((N_USER, 8), jnp.float32),
    ],
    mesh=_MESH,
    compiler_params=pltpu.CompilerParams(use_tc_tiling_on_sc=False),
    scratch_types=[
        pltpu.VMEM((CHUNKS_PER_TILE, EDGE_CHUNK), jnp.int32),
        pltpu.VMEM((CHUNKS_PER_TILE, EDGE_CHUNK), jnp.int32),
        pltpu.VMEM((EDGE_CHUNK, F_OUT), jnp.float32),
        pltpu.VMEM((EDGE_CHUNK, F_OUT), jnp.float32),
        pltpu.VMEM((EDGE_CHUNK, 8), jnp.float32),
        pltpu.VMEM_SHARED((N_USER, F_OUT), jnp.float32),
        pltpu.VMEM_SHARED((N_USER, 8), jnp.float32),
        pltpu.SemaphoreType.DMA,
        pltpu.SemaphoreType.DMA,
    ],
)
def _seg_sum_l1(x_user_h, x_item_h, src_ui, dst_ui, src_iu, dst_iu,
                zeros_64, zeros_8, ones_8,
                agg_item_h, agg_user_h, cnt_item, cnt_user,
                idx_s, idx_d, rows0, rows1, ones_v, acc, cntacc, sem0, sem1):
  c = lax.axis_index("c")
  s = lax.axis_index("s")

  pltpu.sync_copy(ones_8, ones_v)

  def phase(x_h, src_r, dst_r, agg_out, cnt_out, cnt_core):
    _stripe_copy(s, zeros_64, acc)
    _stripe_copy(s, zeros_8, cntacc)
    plsc.subcore_barrier()

    def run(x_view, do_cnt):
      pltpu.sync_copy(src_r.at[s], idx_s)
      pltpu.sync_copy(dst_r.at[s], idx_d)
      _edge_loop(x_view, idx_s, idx_d, rows0, rows1, sem0, sem1, zeros_64,
                 acc, cntacc, ones_v, do_cnt)

    @pl.when(c == 0)
    def _():
      run(x_h.at[0], cnt_core == 0)

    @pl.when(c != 0)
    def _():
      run(x_h.at[1], cnt_core == 1)

    plsc.subcore_barrier()
    _stripe_copy(s, acc, agg_out.at[c])

    @pl.when(c == cnt_core)
    def _():
      _stripe_copy(s, cntacc, cnt_out)

  phase(x_user_h, src_ui, dst_ui, agg_item_h, cnt_item, 0)
  plsc.subcore_barrier()
  phase(x_item_h, src_iu, dst_iu, agg_user_h, cnt_user, 1)


@functools.partial(
    pl.kernel,
    out_type=[jax.ShapeDtypeStruct((N_USER, F_OUT), jnp.float32)
              for _ in range(2)],
    mesh=_MESH,
    compiler_params=pltpu.CompilerParams(use_tc_tiling_on_sc=False),
    scratch_types=[
        pltpu.VMEM((CHUNKS_PER_TILE, EDGE_CHUNK), jnp.int32),
        pltpu.VMEM((CHUNKS_PER_TILE, EDGE_CHUNK), jnp.int32),
        pltpu.VMEM((EDGE_CHUNK, F_OUT), jnp.float32),
        pltpu.VMEM((EDGE_CHUNK, F_OUT), jnp.float32),
        pltpu.VMEM_SHARED((N_USER, F_OUT), jnp.float32),
        pltpu.SemaphoreType.DMA,
        pltpu.SemaphoreType.DMA,
    ],
)
def _seg_sum_l2(q_u, q_i, src_ui, dst_ui, src_iu, dst_iu, zeros_64,
                agg2_item, agg2_user,
                idx_s, idx_d, rows0, rows1, acc, sem0, sem1):
  c = lax.axis_index("c")
  s = lax.axis_index("s")

  _stripe_copy(s, zeros_64, acc)
  plsc.subcore_barrier()

  def process(x_hbm, src_r, dst_r):
    pltpu.sync_copy(src_r.at[s], idx_s)
    pltpu.sync_copy(dst_r.at[s], idx_d)
    _edge_loop(x_hbm, idx_s, idx_d, rows0, rows1, sem0, sem1, zeros_64,
               acc, None, None, False)

  @pl.when(c == 0)
  def _():
    process(q_u, src_ui, dst_ui)

  @pl.when(c != 0)
  def _():
    process(q_i, src_iu, dst_iu)

  plsc.subcore_barrier()

  @pl.when(c == 0)
  def _():
    _stripe_copy(s, acc, agg2_item)

  @pl.when(c != 0)
  def _():
    _stripe_copy(s, acc, agg2_user)


@functools.partial(
    pl.kernel,
    out_type=[jax.ShapeDtypeStruct((LBL_PAD, F_OUT), jnp.float32)
              for _ in range(2)],
    mesh=_MESH,
    compiler_params=pltpu.CompilerParams(use_tc_tiling_on_sc=False),
    scratch_types=[
        pltpu.VMEM((LBL_CHUNKS_PER_TILE, LBL_CHUNK), jnp.int32),
        pltpu.VMEM((LBL_CHUNKS_PER_TILE, LBL_CHUNK), jnp.int32),
        pltpu.VMEM((LBL_CHUNK, F_OUT), jnp.float32),
        pltpu.VMEM((LBL_CHUNK, F_OUT), jnp.float32),
        pltpu.VMEM((LBL_CHUNK, F_OUT), jnp.float32),
        pltpu.VMEM((LBL_CHUNK, F_OUT), jnp.float32),
        pltpu.SemaphoreType.DMA,
        pltpu.SemaphoreType.DMA,
        pltpu.SemaphoreType.DMA,
        pltpu.SemaphoreType.DMA,
    ],
)
def _pair_gather(z_user, z_item, el0_r, el1_r, f1_out, f2_out,
                 idx0, idx1, a0, b0, a1, b1, semA0, semB0, semA1, semB1):
  c = lax.axis_index("c")
  s = lax.axis_index("s")
  w = s * NC + c
  base = w * LBL_CHUNKS_PER_TILE
  pltpu.sync_copy(el0_r.at[w], idx0)
  pltpu.sync_copy(el1_r.at[w], idx1)

  drain = z_user.at[pl.ds(0, LBL_CHUNK)]

  def issue(j, a, b, semA, semB):
    pltpu.async_copy(z_user.at[idx0.at[j]], a, semA)
    pltpu.async_copy(z_item.at[idx1.at[j]], b, semB)

  def write(j, a, b, semA, semB):
    out_base = (base + j) * LBL_CHUNK
    pltpu.make_async_copy(drain, a, semA).wait()
    pltpu.sync_copy(a, f1_out.at[pl.ds(out_base, LBL_CHUNK)])
    pltpu.make_async_copy(drain, b, semB).wait()
    pltpu.sync_copy(b, f2_out.at[pl.ds(out_base, LBL_CHUNK)])

  issue(0, a0, b0, semA0, semB0)

  def pair(i, carry):
    j0 = 2 * i
    issue(j0 + 1, a1, b1, semA1, semB1)
    write(j0, a0, b0, semA0, semB0)
    issue(j0 + 2, a0, b0, semA0, semB0)
    write(j0 + 1, a1, b1, semA1, semB1)
    return carry

  lax.fori_loop(0, LBL_CHUNKS_PER_TILE // 2, pair, 0)
  write(LBL_CHUNKS_PER_TILE - 1, a0, b0, semA0, semB0)


def _dotT(x, w):
  return lax.dot_general(x, w, (((1,), (1,)), ((), ())),
                         preferred_element_type=jnp.float32)


_ROWS_BLK = 1000
_GRID = N_USER // _ROWS_BLK


def _blk(width):
  return pl.BlockSpec((_ROWS_BLK, width), lambda i: (i, 0))


def _full(shape):
  return pl.BlockSpec(shape, lambda i: tuple(0 for _ in shape))


def _hidden_tc_body(agg_i_lo, agg_i_hi, cnt_i_ref, x_i_ref,
                    agg_u_lo, agg_u_hi, cnt_u_ref, x_u_ref,
                    w1l_ui, w1r_ui, b1_ui, w1l_iu, w1r_iu, b1_iu,
                    w2l_ui, w2l_iu, w2r_ui, b2_ui, w2r_iu, b2_iu,
                    q_u_ref, q_i_ref, r2_i_ref, r2_u_ref):
  inv_i = 1.0 / jnp.maximum(cnt_i_ref[:, 0:1], 1.0)
  inv_u = 1.0 / jnp.maximum(cnt_u_ref[:, 0:1], 1.0)
  agg_i = jnp.concatenate([agg_i_lo[...], agg_i_hi[...]], axis=1)
  agg_u = jnp.concatenate([agg_u_lo[...], agg_u_hi[...]], axis=1)
  h_item = jax.nn.relu(_dotT(agg_i * inv_i, w1l_ui[...]) +
                       _dotT(x_i_ref[...], w1r_ui[...]) + b1_ui[...])
  h_user = jax.nn.relu(_dotT(agg_u * inv_u, w1l_iu[...]) +
                       _dotT(x_u_ref[...], w1r_iu[...]) + b1_iu[...])
  q_u_ref[...] = _dotT(h_user, w2l_ui[...])
  q_i_ref[...] = _dotT(h_item, w2l_iu[...])
  r2_i_ref[...] = _dotT(h_item, w2r_ui[...]) + b2_ui[...]
  r2_u_ref[...] = _dotT(h_user, w2r_iu[...]) + b2_iu[...]


_hidden_tc = pl.pallas_call(
    _hidden_tc_body,
    grid=(_GRID,),
    in_specs=[_blk(F_OUT), _blk(F_OUT), _blk(8), _blk(F_IN),
              _blk(F_OUT), _blk(F_OUT), _blk(8), _blk(F_IN),
              _full((F_HID, F_IN)), _full((F_HID, F_IN)), _full((1, F_HID)),
              _full((F_HID, F_IN)), _full((F_HID, F_IN)), _full((1, F_HID)),
              _full((F_OUT, F_HID)), _full((F_OUT, F_HID)),
              _full((F_OUT, F_HID)), _full((1, F_OUT)),
              _full((F_OUT, F_HID)), _full((1, F_OUT))],
    out_specs=[_blk(F_OUT)] * 4,
    out_shape=[jax.ShapeDtypeStruct((N_USER, F_OUT), jnp.float32)] * 4,
)


def _final_tc_body(agg2_i_ref, cnt_i_ref, r2_i_ref, agg2_u_ref, cnt_u_ref,
                   r2_u_ref, z_i_ref, z_u_ref):
  inv_i = 1.0 / jnp.maximum(cnt_i_ref[:, 0:1], 1.0)
  inv_u = 1.0 / jnp.maximum(cnt_u_ref[:, 0:1], 1.0)
  z_i_ref[...] = agg2_i_ref[...] * inv_i + r2_i_ref[...]
  z_u_ref[...] = agg2_u_ref[...] * inv_u + r2_u_ref[...]


_final_tc = pl.pallas_call(
    _final_tc_body,
    grid=(_GRID,),
    in_specs=[_blk(F_OUT), _blk(8), _blk(F_OUT)] * 2,
    out_specs=[_blk(F_OUT)] * 2,
    out_shape=[jax.ShapeDtypeStruct((N_USER, F_OUT), jnp.float32)] * 2,
)


_DOT_BLK = 1024


def _pair_dot_body(f1_ref, f2_ref, out_ref):
  out_ref[...] = jnp.sum(f1_ref[...] * f2_ref[...], axis=1)


_pair_dot_tc = pl.pallas_call(
    _pair_dot_body,
    grid=(LBL_PAD // _DOT_BLK,),
    in_specs=[pl.BlockSpec((_DOT_BLK, F_OUT), lambda i: (i, 0))] * 2,
    out_specs=pl.BlockSpec((_DOT_BLK,), lambda i: (i,)),
    out_shape=jax.ShapeDtypeStruct((LBL_PAD,), jnp.float32),
)


@jax.jit
def kernel(x_user, x_item, edge_index_u2i, edge_index_i2u, edge_label_index,
           W1_l_u2i, b1_u2i, W1_r_u2i, W1_l_i2u, b1_i2u, W1_r_i2u,
           W2_l_u2i, b2_u2i, W2_r_u2i, W2_l_i2u, b2_i2u, W2_r_i2u):
  eshape = (NS, CHUNKS_PER_TILE, EDGE_CHUNK)
  src_ui = edge_index_u2i[0].astype(jnp.int32).reshape(eshape)
  dst_ui = edge_index_u2i[1].astype(jnp.int32).reshape(eshape)
  src_iu = edge_index_i2u[0].astype(jnp.int32).reshape(eshape)
  dst_iu = edge_index_i2u[1].astype(jnp.int32).reshape(eshape)

  zeros_64 = jnp.zeros((N_USER, F_OUT), jnp.float32)
  zeros_8 = jnp.zeros((N_USER, 8), jnp.float32)
  ones_8 = jnp.ones((EDGE_CHUNK, 8), jnp.float32)

  x_user_h = x_user.reshape(N_USER, NC, F_OUT).transpose(1, 0, 2)
  x_item_h = x_item.reshape(N_ITEM, NC, F_OUT).transpose(1, 0, 2)

  agg_item_h, agg_user_h, cnt_item, cnt_user = _seg_sum_l1(
      x_user_h, x_item_h, src_ui, dst_ui, src_iu, dst_iu,
      zeros_64, zeros_8, ones_8)

  q_u, q_i, r2_item, r2_user = _hidden_tc(
      agg_item_h[0], agg_item_h[1], cnt_item, x_item,
      agg_user_h[0], agg_user_h[1], cnt_user, x_user,
      W1_l_u2i, W1_r_u2i, b1_u2i.reshape(1, F_HID),
      W1_l_i2u, W1_r_i2u, b1_i2u.reshape(1, F_HID),
      W2_l_u2i, W2_l_i2u, W2_r_u2i, b2_u2i.reshape(1, F_OUT),
      W2_r_i2u, b2_i2u.reshape(1, F_OUT))

  agg2_item, agg2_user = _seg_sum_l2(
      q_u, q_i, src_ui, dst_ui, src_iu, dst_iu, zeros_64)

  z_item, z_user = _final_tc(agg2_item, cnt_item, r2_item,
                             agg2_user, cnt_user, r2_user)

  el = edge_label_index.astype(jnp.int32)
  pad = LBL_PAD - E_LBL
  lshape = (NC * NS, LBL_CHUNKS_PER_TILE, LBL_CHUNK)
  el0_r = jnp.concatenate([el[0], jnp.zeros((pad,), jnp.int32)]).reshape(lshape)
  el1_r = jnp.concatenate([el[1], jnp.zeros((pad,), jnp.int32)]).reshape(lshape)

  f1, f2 = _pair_gather(z_user, z_item, el0_r, el1_r)
  pred = _pair_dot_tc(f1, f2)
  return pred[:E_LBL]


# speedup vs baseline: 6.5820x; 1.0048x over previous
import functools

import jax
import jax.numpy as jnp
from jax import lax
from jax.experimental import pallas as pl
from jax.experimental.pallas import tpu as pltpu
from jax.experimental.pallas import tpu_sc as plsc

N_USER = 10000
N_ITEM = 10000
E = 320000
E_LBL = 100000
F_IN = 128
F_HID = 128
F_OUT = 64

NC = 2
NS = 16

EDGE_CHUNK = 80
EDGES_PER_TILE = E // NS
CHUNKS_PER_TILE = EDGES_PER_TILE // EDGE_CHUNK
CHUNK_PAIRS = CHUNKS_PER_TILE // 2
N_CHUNK_ROWS = E // EDGE_CHUNK

STRIPE = 624
TAIL_BASE = STRIPE * NS
TAIL = N_USER - TAIL_BASE

LBL_PAD = 102400
LBL_CHUNK = 128
LBL_CHUNKS_PER_TILE = LBL_PAD // (NC * NS) // LBL_CHUNK

_MESH = plsc.VectorSubcoreMesh(core_axis_name="c", subcore_axis_name="s")


def _edge_loop(x_view, idx_s, idx_d, rows0, rows1, sem0, sem1, drain_hbm,
               acc, hist, do_cnt):
  drain = drain_hbm.at[pl.ds(0, EDGE_CHUNK)]
  ones16 = jnp.full((16,), 1.0, jnp.float32)

  def scatter(rows, j):
    pltpu.sync_copy(rows, acc.at[idx_d.at[j]], add=True)
    if do_cnt:
      for k in range(EDGE_CHUNK // 16):
        idx16 = idx_d[j, pl.ds(16 * k, 16)]
        plsc.addupdate_scatter(
            hist, [lax.shift_right_logical(idx16, 3),
                   lax.bitwise_and(idx16, 7)], ones16)

  pltpu.async_copy(x_view.at[idx_s.at[0]], rows0, sem0)

  def pair(i, carry):
    j0 = 2 * i
    pltpu.make_async_copy(drain, rows0, sem0).wait()
    pltpu.async_copy(x_view.at[idx_s.at[j0 + 1]], rows1, sem1)
    scatter(rows0, j0)

    @pl.when(i < CHUNK_PAIRS - 1)
    def _():
      pltpu.async_copy(x_view.at[idx_s.at[j0 + 2]], rows0, sem0)

    pltpu.make_async_copy(drain, rows1, sem1).wait()
    scatter(rows1, j0 + 1)
    return carry

  lax.fori_loop(0, CHUNK_PAIRS, pair, 0)


def _stripe_copy(s, src, dst):
  r0 = s * STRIPE
  pltpu.sync_copy(src.at[pl.ds(r0, STRIPE)], dst.at[pl.ds(r0, STRIPE)])

  @pl.when(s == NS - 1)
  def _():
    pltpu.sync_copy(src.at[pl.ds(TAIL_BASE, TAIL)],
                    dst.at[pl.ds(TAIL_BASE, TAIL)])


def _cnt_stripe_copy(s, src, dst):
  @pl.when(s < NS - 1)
  def _():
    pltpu.sync_copy(src.at[pl.ds(s * 80, 80)], dst.at[pl.ds(s * 80, 80)])

  @pl.when(s == NS - 1)
  def _():
    pltpu.sync_copy(src.at[pl.ds(1200, 50)], dst.at[pl.ds(1200, 50)])


@functools.partial(
    pl.kernel,
    out_type=[
        jax.ShapeDtypeStruct((NC, N_USER, F_OUT), jnp.float32),
        jax.ShapeDtypeStruct((NC, N_USER, F_OUT), jnp.float32),
        jax.ShapeDtypeStruct((1250, 8), jnp.float32),
        jax.ShapeDtypeStruct((1250, 8), jnp.float32),
    ],
    mesh=_MESH,
    compiler_params=pltpu.CompilerParams(use_tc_tiling_on_sc=False,
                                         needs_layout_passes=False),
    scratch_types=[
        pltpu.VMEM((CHUNKS_PER_TILE, EDGE_CHUNK), jnp.int32),
        pltpu.VMEM((CHUNKS_PER_TILE, EDGE_CHUNK), jnp.int32),
        pltpu.VMEM((EDGE_CHUNK, F_OUT), jnp.float32),
        pltpu.VMEM((EDGE_CHUNK, F_OUT), jnp.float32),
        pltpu.VMEM((1250, 8), jnp.float32),
        pltpu.VMEM((10, 125), jnp.int32),
        pltpu.VMEM_SHARED((N_USER, F_OUT), jnp.float32),
        pltpu.VMEM_SHARED((1250, 8), jnp.float32),
        pltpu.SemaphoreType.DMA,
        pltpu.SemaphoreType.DMA,
    ],
)
def _seg_sum_l1(x_user_h, x_item_h, src_ui, dst_ui, src_iu, dst_iu,
                zeros_64, zeros_cnt, iota_r,
                agg_item_h, agg_user_h, cnt_item, cnt_user,
                idx_s, idx_d, rows0, rows1, hist, iota_v, acc, cntacc,
                sem0, sem1):
  c = lax.axis_index("c")
  s = lax.axis_index("s")
  pltpu.sync_copy(iota_r, iota_v)

  def phase(x_h, src_r, dst_r, agg_out, cnt_out, cnt_core):
    _stripe_copy(s, zeros_64, acc)
    _cnt_stripe_copy(s, zeros_cnt, cntacc)
    plsc.subcore_barrier()

    def run(x_view, do_cnt):
      pltpu.sync_copy(src_r.at[s], idx_s)
      pltpu.sync_copy(dst_r.at[s], idx_d)
      if do_cnt:
        pltpu.sync_copy(zeros_cnt, hist)
      _edge_loop(x_view, idx_s, idx_d, rows0, rows1, sem0, sem1, zeros_64,
                 acc, hist, do_cnt)
      if do_cnt:
        for t in range(10):
          pltpu.sync_copy(hist.at[pl.ds(t * 125, 125)],
                          cntacc.at[iota_v.at[t]], add=True)

    @pl.when(c == 0)
    def _():
      run(x_h.at[0], cnt_core == 0)

    @pl.when(c != 0)
    def _():
      run(x_h.at[1], cnt_core == 1)

    plsc.subcore_barrier()
    _stripe_copy(s, acc, agg_out.at[c])

    @pl.when(c == cnt_core)
    def _():
      _cnt_stripe_copy(s, cntacc, cnt_out)

  phase(x_user_h, src_ui, dst_ui, agg_item_h, cnt_item, 0)
  plsc.subcore_barrier()
  phase(x_item_h, src_iu, dst_iu, agg_user_h, cnt_user, 1)


@functools.partial(
    pl.kernel,
    out_type=[jax.ShapeDtypeStruct((N_USER, F_OUT), jnp.float32)
              for _ in range(2)],
    mesh=_MESH,
    compiler_params=pltpu.CompilerParams(use_tc_tiling_on_sc=False),
    scratch_types=[
        pltpu.VMEM((CHUNKS_PER_TILE, EDGE_CHUNK), jnp.int32),
        pltpu.VMEM((CHUNKS_PER_TILE, EDGE_CHUNK), jnp.int32),
        pltpu.VMEM((EDGE_CHUNK, F_OUT), jnp.float32),
        pltpu.VMEM((EDGE_CHUNK, F_OUT), jnp.float32),
        pltpu.VMEM_SHARED((N_USER, F_OUT), jnp.float32),
        pltpu.SemaphoreType.DMA,
        pltpu.SemaphoreType.DMA,
    ],
)
def _seg_sum_l2(q_u, q_i, src_ui, dst_ui, src_iu, dst_iu, zeros_64,
                agg2_item, agg2_user,
                idx_s, idx_d, rows0, rows1, acc, sem0, sem1):
  c = lax.axis_index("c")
  s = lax.axis_index("s")

  _stripe_copy(s, zeros_64, acc)
  plsc.subcore_barrier()

  def process(x_hbm, src_r, dst_r):
    pltpu.sync_copy(src_r.at[s], idx_s)
    pltpu.sync_copy(dst_r.at[s], idx_d)
    _edge_loop(x_hbm, idx_s, idx_d, rows0, rows1, sem0, sem1, zeros_64,
               acc, None, False)

  @pl.when(c == 0)
  def _():
    process(q_u, src_ui, dst_ui)

  @pl.when(c != 0)
  def _():
    process(q_i, src_iu, dst_iu)

  plsc.subcore_barrier()

  @pl.when(c == 0)
  def _():
    _stripe_copy(s, acc, agg2_item)

  @pl.when(c != 0)
  def _():
    _stripe_copy(s, acc, agg2_user)


@functools.partial(
    pl.kernel,
    out_type=[jax.ShapeDtypeStruct((LBL_PAD, F_OUT), jnp.float32)
              for _ in range(2)],
    mesh=_MESH,
    compiler_params=pltpu.CompilerParams(use_tc_tiling_on_sc=False),
    scratch_types=[
        pltpu.VMEM((LBL_CHUNKS_PER_TILE, LBL_CHUNK), jnp.int32),
        pltpu.VMEM((LBL_CHUNKS_PER_TILE, LBL_CHUNK), jnp.int32),
        pltpu.VMEM((LBL_CHUNK, F_OUT), jnp.float32),
        pltpu.VMEM((LBL_CHUNK, F_OUT), jnp.float32),
        pltpu.VMEM((LBL_CHUNK, F_OUT), jnp.float32),
        pltpu.VMEM((LBL_CHUNK, F_OUT), jnp.float32),
        pltpu.SemaphoreType.DMA,
        pltpu.SemaphoreType.DMA,
        pltpu.SemaphoreType.DMA,
        pltpu.SemaphoreType.DMA,
    ],
)
def _pair_gather(z_user, z_item, el0_r, el1_r, f1_out, f2_out,
                 idx0, idx1, a0, b0, a1, b1, semA0, semB0, semA1, semB1):
  c = lax.axis_index("c")
  s = lax.axis_index("s")
  w = s * NC + c
  base = w * LBL_CHUNKS_PER_TILE
  pltpu.sync_copy(el0_r.at[w], idx0)
  pltpu.sync_copy(el1_r.at[w], idx1)

  drain = z_user.at[pl.ds(0, LBL_CHUNK)]

  def issue(j, a, b, semA, semB):
    pltpu.async_copy(z_user.at[idx0.at[j]], a, semA)
    pltpu.async_copy(z_item.at[idx1.at[j]], b, semB)

  def write(j, a, b, semA, semB):
    out_base = (base + j) * LBL_CHUNK
    pltpu.make_async_copy(drain, a, semA).wait()
    pltpu.sync_copy(a, f1_out.at[pl.ds(out_base, LBL_CHUNK)])
    pltpu.make_async_copy(drain, b, semB).wait()
    pltpu.sync_copy(b, f2_out.at[pl.ds(out_base, LBL_CHUNK)])

  issue(0, a0, b0, semA0, semB0)

  def pair(i, carry):
    j0 = 2 * i
    issue(j0 + 1, a1, b1, semA1, semB1)
    write(j0, a0, b0, semA0, semB0)
    issue(j0 + 2, a0, b0, semA0, semB0)
    write(j0 + 1, a1, b1, semA1, semB1)
    return carry

  lax.fori_loop(0, LBL_CHUNKS_PER_TILE // 2, pair, 0)
  write(LBL_CHUNKS_PER_TILE - 1, a0, b0, semA0, semB0)


def _dotT(x, w):
  return lax.dot_general(x, w, (((1,), (1,)), ((), ())),
                         preferred_element_type=jnp.float32)


_ROWS_BLK = 1000
_GRID = N_USER // _ROWS_BLK


def _blk(width):
  return pl.BlockSpec((_ROWS_BLK, width), lambda i: (i, 0))


def _full(shape):
  return pl.BlockSpec(shape, lambda i: tuple(0 for _ in shape))


def _hidden_tc_body(agg_i_lo, agg_i_hi, cnt_i_ref, x_i_ref,
                    agg_u_lo, agg_u_hi, cnt_u_ref, x_u_ref,
                    w1l_ui, w1r_ui, b1_ui, w1l_iu, w1r_iu, b1_iu,
                    w2l_ui, w2l_iu, w2r_ui, b2_ui, w2r_iu, b2_iu,
                    q_u_ref, q_i_ref, r2_i_ref, r2_u_ref):
  inv_i = 1.0 / jnp.maximum(cnt_i_ref[...], 1.0)
  inv_u = 1.0 / jnp.maximum(cnt_u_ref[...], 1.0)
  agg_i = jnp.concatenate([agg_i_lo[...], agg_i_hi[...]], axis=1)
  agg_u = jnp.concatenate([agg_u_lo[...], agg_u_hi[...]], axis=1)
  h_item = jax.nn.relu(_dotT(agg_i * inv_i, w1l_ui[...]) +
                       _dotT(x_i_ref[...], w1r_ui[...]) + b1_ui[...])
  h_user = jax.nn.relu(_dotT(agg_u * inv_u, w1l_iu[...]) +
                       _dotT(x_u_ref[...], w1r_iu[...]) + b1_iu[...])
  q_u_ref[...] = _dotT(h_user, w2l_ui[...])
  q_i_ref[...] = _dotT(h_item, w2l_iu[...])
  r2_i_ref[...] = _dotT(h_item, w2r_ui[...]) + b2_ui[...]
  r2_u_ref[...] = _dotT(h_user, w2r_iu[...]) + b2_iu[...]


_hidden_tc = pl.pallas_call(
    _hidden_tc_body,
    grid=(_GRID,),
    in_specs=[_blk(F_OUT), _blk(F_OUT), _blk(1), _blk(F_IN),
              _blk(F_OUT), _blk(F_OUT), _blk(1), _blk(F_IN),
              _full((F_HID, F_IN)), _full((F_HID, F_IN)), _full((1, F_HID)),
              _full((F_HID, F_IN)), _full((F_HID, F_IN)), _full((1, F_HID)),
              _full((F_OUT, F_HID)), _full((F_OUT, F_HID)),
              _full((F_OUT, F_HID)), _full((1, F_OUT)),
              _full((F_OUT, F_HID)), _full((1, F_OUT))],
    out_specs=[_blk(F_OUT)] * 4,
    out_shape=[jax.ShapeDtypeStruct((N_USER, F_OUT), jnp.float32)] * 4,
)


def _final_tc_body(agg2_i_ref, cnt_i_ref, r2_i_ref, agg2_u_ref, cnt_u_ref,
                   r2_u_ref, z_i_ref, z_u_ref):
  inv_i = 1.0 / jnp.maximum(cnt_i_ref[...], 1.0)
  inv_u = 1.0 / jnp.maximum(cnt_u_ref[...], 1.0)
  z_i_ref[...] = agg2_i_ref[...] * inv_i + r2_i_ref[...]
  z_u_ref[...] = agg2_u_ref[...] * inv_u + r2_u_ref[...]


_final_tc = pl.pallas_call(
    _final_tc_body,
    grid=(_GRID,),
    in_specs=[_blk(F_OUT), _blk(1), _blk(F_OUT)] * 2,
    out_specs=[_blk(F_OUT)] * 2,
    out_shape=[jax.ShapeDtypeStruct((N_USER, F_OUT), jnp.float32)] * 2,
)


_DOT_BLK = 1024


def _pair_dot_body(f1_ref, f2_ref, out_ref):
  out_ref[...] = jnp.sum(f1_ref[...] * f2_ref[...], axis=1)


_pair_dot_tc = pl.pallas_call(
    _pair_dot_body,
    grid=(LBL_PAD // _DOT_BLK,),
    in_specs=[pl.BlockSpec((_DOT_BLK, F_OUT), lambda i: (i, 0))] * 2,
    out_specs=pl.BlockSpec((_DOT_BLK,), lambda i: (i,)),
    out_shape=jax.ShapeDtypeStruct((LBL_PAD,), jnp.float32),
)


@jax.jit
def kernel(x_user, x_item, edge_index_u2i, edge_index_i2u, edge_label_index,
           W1_l_u2i, b1_u2i, W1_r_u2i, W1_l_i2u, b1_i2u, W1_r_i2u,
           W2_l_u2i, b2_u2i, W2_r_u2i, W2_l_i2u, b2_i2u, W2_r_i2u):
  eshape = (NS, CHUNKS_PER_TILE, EDGE_CHUNK)
  src_ui = edge_index_u2i[0].astype(jnp.int32).reshape(eshape)
  dst_ui = edge_index_u2i[1].astype(jnp.int32).reshape(eshape)
  src_iu = edge_index_i2u[0].astype(jnp.int32).reshape(eshape)
  dst_iu = edge_index_i2u[1].astype(jnp.int32).reshape(eshape)

  zeros_64 = jnp.zeros((N_USER, F_OUT), jnp.float32)
  zeros_cnt = jnp.zeros((1250, 8), jnp.float32)
  iota_r = jnp.arange(1250, dtype=jnp.int32).reshape(10, 125)

  x_user_h = x_user.reshape(N_USER, NC, F_OUT).transpose(1, 0, 2)
  x_item_h = x_item.reshape(N_ITEM, NC, F_OUT).transpose(1, 0, 2)

  agg_item_h, agg_user_h, cnt_item_p, cnt_user_p = _seg_sum_l1(
      x_user_h, x_item_h, src_ui, dst_ui, src_iu, dst_iu,
      zeros_64, zeros_cnt, iota_r)
  cnt_item = cnt_item_p.reshape(N_ITEM, 1)
  cnt_user = cnt_user_p.reshape(N_USER, 1)

  q_u, q_i, r2_item, r2_user = _hidden_tc(
      agg_item_h[0], agg_item_h[1], cnt_item, x_item,
      agg_user_h[0], agg_user_h[1], cnt_user, x_user,
      W1_l_u2i, W1_r_u2i, b1_u2i.reshape(1, F_HID),
      W1_l_i2u, W1_r_i2u, b1_i2u.reshape(1, F_HID),
      W2_l_u2i, W2_l_i2u, W2_r_u2i, b2_u2i.reshape(1, F_OUT),
      W2_r_i2u, b2_i2u.reshape(1, F_OUT))

  agg2_item, agg2_user = _seg_sum_l2(
      q_u, q_i, src_ui, dst_ui, src_iu, dst_iu, zeros_64)

  z_item, z_user = _final_tc(agg2_item, cnt_item, r2_item,
                             agg2_user, cnt_user, r2_user)

  el = edge_label_index.astype(jnp.int32)
  pad = LBL_PAD - E_LBL
  lshape = (NC * NS, LBL_CHUNKS_PER_TILE, LBL_CHUNK)
  el0_r = jnp.concatenate([el[0], jnp.zeros((pad,), jnp.int32)]).reshape(lshape)
  el1_r = jnp.concatenate([el[1], jnp.zeros((pad,), jnp.int32)]).reshape(lshape)

  f1, f2 = _pair_gather(z_user, z_item, el0_r, el1_r)
  pred = _pair_dot_tc(f1, f2)
  return pred[:E_LBL]


# speedup vs baseline: 7.2351x; 1.0992x over previous
import functools

import jax
import jax.numpy as jnp
from jax import lax
from jax.experimental import pallas as pl
from jax.experimental.pallas import tpu as pltpu
from jax.experimental.pallas import tpu_sc as plsc

N_USER = 10000
N_ITEM = 10000
E = 320000
E_LBL = 100000
F_IN = 128
F_HID = 128
F_OUT = 64

NC = 2
NS = 16

EDGE_CHUNK = 80
EDGES_PER_TILE = E // NS
CHUNKS_PER_TILE = EDGES_PER_TILE // EDGE_CHUNK
CHUNK_PAIRS = CHUNKS_PER_TILE // 2
N_CHUNK_ROWS = E // EDGE_CHUNK

STRIPE = 624
TAIL_BASE = STRIPE * NS
TAIL = N_USER - TAIL_BASE

LBL_PAD = 102400
LBL_CHUNK = 128
LBL_CHUNKS_PER_TILE = LBL_PAD // (NC * NS) // LBL_CHUNK

_MESH = plsc.VectorSubcoreMesh(core_axis_name="c", subcore_axis_name="s")


def _edge_loop(x_view, idx_s, idx_d, rows0, rows1, sem0, sem1, drain_hbm,
               acc, hist, do_cnt):
  drain = drain_hbm.at[pl.ds(0, EDGE_CHUNK)]
  ones16 = jnp.full((16,), 1.0, jnp.float32)

  def scatter(rows, j):
    pltpu.sync_copy(rows, acc.at[idx_d.at[j]], add=True)
    if do_cnt:
      for k in range(EDGE_CHUNK // 16):
        idx16 = idx_d[j, pl.ds(16 * k, 16)]
        plsc.addupdate_scatter(
            hist, [lax.shift_right_logical(idx16, 3),
                   lax.bitwise_and(idx16, 7)], ones16)

  pltpu.async_copy(x_view.at[idx_s.at[0]], rows0, sem0)

  def pair(i, carry):
    j0 = 2 * i
    pltpu.make_async_copy(drain, rows0, sem0).wait()
    pltpu.async_copy(x_view.at[idx_s.at[j0 + 1]], rows1, sem1)
    scatter(rows0, j0)

    @pl.when(i < CHUNK_PAIRS - 1)
    def _():
      pltpu.async_copy(x_view.at[idx_s.at[j0 + 2]], rows0, sem0)

    pltpu.make_async_copy(drain, rows1, sem1).wait()
    scatter(rows1, j0 + 1)
    return carry

  lax.fori_loop(0, CHUNK_PAIRS, pair, 0)


def _stripe_copy(s, src, dst):
  r0 = s * STRIPE
  pltpu.sync_copy(src.at[pl.ds(r0, STRIPE)], dst.at[pl.ds(r0, STRIPE)])

  @pl.when(s == NS - 1)
  def _():
    pltpu.sync_copy(src.at[pl.ds(TAIL_BASE, TAIL)],
                    dst.at[pl.ds(TAIL_BASE, TAIL)])


def _cnt_stripe_copy(s, src, dst):
  @pl.when(s < NS - 1)
  def _():
    pltpu.sync_copy(src.at[pl.ds(s * 80, 80)], dst.at[pl.ds(s * 80, 80)])

  @pl.when(s == NS - 1)
  def _():
    pltpu.sync_copy(src.at[pl.ds(1200, 50)], dst.at[pl.ds(1200, 50)])


@functools.partial(
    pl.kernel,
    out_type=[
        jax.ShapeDtypeStruct((NC, N_USER, F_OUT), jnp.float32),
        jax.ShapeDtypeStruct((NC, N_USER, F_OUT), jnp.float32),
        jax.ShapeDtypeStruct((1250, 8), jnp.float32),
        jax.ShapeDtypeStruct((1250, 8), jnp.float32),
    ],
    mesh=_MESH,
    compiler_params=pltpu.CompilerParams(use_tc_tiling_on_sc=False,
                                         needs_layout_passes=False),
    scratch_types=[
        pltpu.VMEM((CHUNKS_PER_TILE, EDGE_CHUNK), jnp.int32),
        pltpu.VMEM((CHUNKS_PER_TILE, EDGE_CHUNK), jnp.int32),
        pltpu.VMEM((EDGE_CHUNK, F_OUT), jnp.float32),
        pltpu.VMEM((EDGE_CHUNK, F_OUT), jnp.float32),
        pltpu.VMEM((1250, 8), jnp.float32),
        pltpu.VMEM((10, 125), jnp.int32),
        pltpu.VMEM_SHARED((N_USER, F_OUT), jnp.float32),
        pltpu.VMEM_SHARED((1250, 8), jnp.float32),
        pltpu.SemaphoreType.DMA,
        pltpu.SemaphoreType.DMA,
    ],
)
def _seg_sum_l1(x_user_h, x_item_h, src_ui, dst_ui, src_iu, dst_iu,
                zeros_64, zeros_cnt, iota_r,
                agg_item_h, agg_user_h, cnt_item, cnt_user,
                idx_s, idx_d, rows0, rows1, hist, iota_v, acc, cntacc,
                sem0, sem1):
  c = lax.axis_index("c")
  s = lax.axis_index("s")
  pltpu.sync_copy(iota_r, iota_v)

  def phase(x_h, src_r, dst_r, agg_out, cnt_out, cnt_core):
    _stripe_copy(s, zeros_64, acc)
    _cnt_stripe_copy(s, zeros_cnt, cntacc)
    plsc.subcore_barrier()

    def run(x_view, do_cnt):
      pltpu.sync_copy(src_r.at[s], idx_s)
      pltpu.sync_copy(dst_r.at[s], idx_d)
      if do_cnt:
        pltpu.sync_copy(zeros_cnt, hist)
      _edge_loop(x_view, idx_s, idx_d, rows0, rows1, sem0, sem1, zeros_64,
                 acc, hist, do_cnt)
      if do_cnt:
        for t in range(10):
          pltpu.sync_copy(hist.at[pl.ds(t * 125, 125)],
                          cntacc.at[iota_v.at[t]], add=True)

    @pl.when(c == 0)
    def _():
      run(x_h.at[0], cnt_core == 0)

    @pl.when(c != 0)
    def _():
      run(x_h.at[1], cnt_core == 1)

    plsc.subcore_barrier()
    _stripe_copy(s, acc, agg_out.at[c])

    @pl.when(c == cnt_core)
    def _():
      _cnt_stripe_copy(s, cntacc, cnt_out)

  phase(x_user_h, src_ui, dst_ui, agg_item_h, cnt_item, 0)
  plsc.subcore_barrier()
  phase(x_item_h, src_iu, dst_iu, agg_user_h, cnt_user, 1)


@functools.partial(
    pl.kernel,
    out_type=[jax.ShapeDtypeStruct((N_USER, F_OUT), jnp.float32)
              for _ in range(2)],
    mesh=_MESH,
    compiler_params=pltpu.CompilerParams(use_tc_tiling_on_sc=False),
    scratch_types=[
        pltpu.VMEM((CHUNKS_PER_TILE, EDGE_CHUNK), jnp.int32),
        pltpu.VMEM((CHUNKS_PER_TILE, EDGE_CHUNK), jnp.int32),
        pltpu.VMEM((EDGE_CHUNK, F_OUT), jnp.float32),
        pltpu.VMEM((EDGE_CHUNK, F_OUT), jnp.float32),
        pltpu.VMEM_SHARED((N_USER, F_OUT), jnp.float32),
        pltpu.SemaphoreType.DMA,
        pltpu.SemaphoreType.DMA,
    ],
)
def _seg_sum_l2(q_u, q_i, src_ui, dst_ui, src_iu, dst_iu, zeros_64,
                agg2_item, agg2_user,
                idx_s, idx_d, rows0, rows1, acc, sem0, sem1):
  c = lax.axis_index("c")
  s = lax.axis_index("s")

  _stripe_copy(s, zeros_64, acc)
  plsc.subcore_barrier()

  def process(x_hbm, src_r, dst_r):
    pltpu.sync_copy(src_r.at[s], idx_s)
    pltpu.sync_copy(dst_r.at[s], idx_d)
    _edge_loop(x_hbm, idx_s, idx_d, rows0, rows1, sem0, sem1, zeros_64,
               acc, None, False)

  @pl.when(c == 0)
  def _():
    process(q_u, src_ui, dst_ui)

  @pl.when(c != 0)
  def _():
    process(q_i, src_iu, dst_iu)

  plsc.subcore_barrier()

  @pl.when(c == 0)
  def _():
    _stripe_copy(s, acc, agg2_item)

  @pl.when(c != 0)
  def _():
    _stripe_copy(s, acc, agg2_user)


@functools.partial(
    pl.kernel,
    out_type=jax.ShapeDtypeStruct((LBL_PAD, 16), jnp.float32),
    mesh=_MESH,
    compiler_params=pltpu.CompilerParams(use_tc_tiling_on_sc=False,
                                         needs_layout_passes=False),
    scratch_types=[
        pltpu.VMEM((LBL_CHUNKS_PER_TILE, LBL_CHUNK), jnp.int32),
        pltpu.VMEM((LBL_CHUNKS_PER_TILE, LBL_CHUNK), jnp.int32),
        pltpu.VMEM((LBL_CHUNK, F_OUT), jnp.float32),
        pltpu.VMEM((LBL_CHUNK, F_OUT), jnp.float32),
        pltpu.VMEM((LBL_CHUNK, F_OUT), jnp.float32),
        pltpu.VMEM((LBL_CHUNK, F_OUT), jnp.float32),
        pltpu.VMEM((LBL_CHUNK, 16), jnp.float32),
        pltpu.SemaphoreType.DMA,
        pltpu.SemaphoreType.DMA,
        pltpu.SemaphoreType.DMA,
        pltpu.SemaphoreType.DMA,
    ],
)
def _pair_gather(z_user, z_item, el0_r, el1_r, pd_out,
                 idx0, idx1, a0, b0, a1, b1, ov,
                 semA0, semB0, semA1, semB1):
  c = lax.axis_index("c")
  s = lax.axis_index("s")
  w = s * NC + c
  base = w * LBL_CHUNKS_PER_TILE
  pltpu.sync_copy(el0_r.at[w], idx0)
  pltpu.sync_copy(el1_r.at[w], idx1)

  drain = z_user.at[pl.ds(0, LBL_CHUNK)]

  def issue(j, a, b, semA, semB):
    pltpu.async_copy(z_user.at[idx0.at[j]], a, semA)
    pltpu.async_copy(z_item.at[idx1.at[j]], b, semB)

  def reduce_write(j, a, b, semA, semB):
    out_base = (base + j) * LBL_CHUNK
    pltpu.make_async_copy(drain, a, semA).wait()
    pltpu.make_async_copy(drain, b, semB).wait()

    def row(r, carry):
      acc = a[r, pl.ds(0, 16)] * b[r, pl.ds(0, 16)]
      for q in range(1, 4):
        acc = acc + a[r, pl.ds(16 * q, 16)] * b[r, pl.ds(16 * q, 16)]
      ov[r, :] = acc
      return carry

    lax.fori_loop(0, LBL_CHUNK, row, 0)
    pltpu.sync_copy(ov, pd_out.at[pl.ds(out_base, LBL_CHUNK)])

  issue(0, a0, b0, semA0, semB0)

  def pair(i, carry):
    j0 = 2 * i
    issue(j0 + 1, a1, b1, semA1, semB1)
    reduce_write(j0, a0, b0, semA0, semB0)
    issue(j0 + 2, a0, b0, semA0, semB0)
    reduce_write(j0 + 1, a1, b1, semA1, semB1)
    return carry

  lax.fori_loop(0, LBL_CHUNKS_PER_TILE // 2, pair, 0)
  reduce_write(LBL_CHUNKS_PER_TILE - 1, a0, b0, semA0, semB0)


def _dotT(x, w):
  return lax.dot_general(x, w, (((1,), (1,)), ((), ())),
                         preferred_element_type=jnp.float32)


_ROWS_BLK = 1000
_GRID = N_USER // _ROWS_BLK


def _blk(width):
  return pl.BlockSpec((_ROWS_BLK, width), lambda i: (i, 0))


def _full(shape):
  return pl.BlockSpec(shape, lambda i: tuple(0 for _ in shape))


def _hidden_tc_body(agg_i_lo, agg_i_hi, cnt_i_ref, x_i_ref,
                    agg_u_lo, agg_u_hi, cnt_u_ref, x_u_ref,
                    w1l_ui, w1r_ui, b1_ui, w1l_iu, w1r_iu, b1_iu,
                    w2l_ui, w2l_iu, w2r_ui, b2_ui, w2r_iu, b2_iu,
                    q_u_ref, q_i_ref, r2_i_ref, r2_u_ref):
  inv_i = 1.0 / jnp.maximum(cnt_i_ref[...], 1.0)
  inv_u = 1.0 / jnp.maximum(cnt_u_ref[...], 1.0)
  agg_i = jnp.concatenate([agg_i_lo[...], agg_i_hi[...]], axis=1)
  agg_u = jnp.concatenate([agg_u_lo[...], agg_u_hi[...]], axis=1)
  h_item = jax.nn.relu(_dotT(agg_i * inv_i, w1l_ui[...]) +
                       _dotT(x_i_ref[...], w1r_ui[...]) + b1_ui[...])
  h_user = jax.nn.relu(_dotT(agg_u * inv_u, w1l_iu[...]) +
                       _dotT(x_u_ref[...], w1r_iu[...]) + b1_iu[...])
  q_u_ref[...] = _dotT(h_user, w2l_ui[...])
  q_i_ref[...] = _dotT(h_item, w2l_iu[...])
  r2_i_ref[...] = _dotT(h_item, w2r_ui[...]) + b2_ui[...]
  r2_u_ref[...] = _dotT(h_user, w2r_iu[...]) + b2_iu[...]


_hidden_tc = pl.pallas_call(
    _hidden_tc_body,
    grid=(_GRID,),
    in_specs=[_blk(F_OUT), _blk(F_OUT), _blk(1), _blk(F_IN),
              _blk(F_OUT), _blk(F_OUT), _blk(1), _blk(F_IN),
              _full((F_HID, F_IN)), _full((F_HID, F_IN)), _full((1, F_HID)),
              _full((F_HID, F_IN)), _full((F_HID, F_IN)), _full((1, F_HID)),
              _full((F_OUT, F_HID)), _full((F_OUT, F_HID)),
              _full((F_OUT, F_HID)), _full((1, F_OUT)),
              _full((F_OUT, F_HID)), _full((1, F_OUT))],
    out_specs=[_blk(F_OUT)] * 4,
    out_shape=[jax.ShapeDtypeStruct((N_USER, F_OUT), jnp.float32)] * 4,
)


def _final_tc_body(agg2_i_ref, cnt_i_ref, r2_i_ref, agg2_u_ref, cnt_u_ref,
                   r2_u_ref, z_i_ref, z_u_ref):
  inv_i = 1.0 / jnp.maximum(cnt_i_ref[...], 1.0)
  inv_u = 1.0 / jnp.maximum(cnt_u_ref[...], 1.0)
  z_i_ref[...] = agg2_i_ref[...] * inv_i + r2_i_ref[...]
  z_u_ref[...] = agg2_u_ref[...] * inv_u + r2_u_ref[...]


_final_tc = pl.pallas_call(
    _final_tc_body,
    grid=(_GRID,),
    in_specs=[_blk(F_OUT), _blk(1), _blk(F_OUT)] * 2,
    out_specs=[_blk(F_OUT)] * 2,
    out_shape=[jax.ShapeDtypeStruct((N_USER, F_OUT), jnp.float32)] * 2,
)


_DOT_BLK = 1024


def _pair_dot_body(pd_ref, out_ref):
  out_ref[...] = jnp.sum(pd_ref[...], axis=1)


_pair_dot_tc = pl.pallas_call(
    _pair_dot_body,
    grid=(LBL_PAD // _DOT_BLK,),
    in_specs=[pl.BlockSpec((_DOT_BLK, 16), lambda i: (i, 0))],
    out_specs=pl.BlockSpec((_DOT_BLK,), lambda i: (i,)),
    out_shape=jax.ShapeDtypeStruct((LBL_PAD,), jnp.float32),
)


@jax.jit
def kernel(x_user, x_item, edge_index_u2i, edge_index_i2u, edge_label_index,
           W1_l_u2i, b1_u2i, W1_r_u2i, W1_l_i2u, b1_i2u, W1_r_i2u,
           W2_l_u2i, b2_u2i, W2_r_u2i, W2_l_i2u, b2_i2u, W2_r_i2u):
  eshape = (NS, CHUNKS_PER_TILE, EDGE_CHUNK)
  src_ui = edge_index_u2i[0].astype(jnp.int32).reshape(eshape)
  dst_ui = edge_index_u2i[1].astype(jnp.int32).reshape(eshape)
  src_iu = edge_index_i2u[0].astype(jnp.int32).reshape(eshape)
  dst_iu = edge_index_i2u[1].astype(jnp.int32).reshape(eshape)

  zeros_64 = jnp.zeros((N_USER, F_OUT), jnp.float32)
  zeros_cnt = jnp.zeros((1250, 8), jnp.float32)
  iota_r = jnp.arange(1250, dtype=jnp.int32).reshape(10, 125)

  x_user_h = x_user.reshape(N_USER, NC, F_OUT).transpose(1, 0, 2)
  x_item_h = x_item.reshape(N_ITEM, NC, F_OUT).transpose(1, 0, 2)

  agg_item_h, agg_user_h, cnt_item_p, cnt_user_p = _seg_sum_l1(
      x_user_h, x_item_h, src_ui, dst_ui, src_iu, dst_iu,
      zeros_64, zeros_cnt, iota_r)
  cnt_item = cnt_item_p.reshape(N_ITEM, 1)
  cnt_user = cnt_user_p.reshape(N_USER, 1)

  q_u, q_i, r2_item, r2_user = _hidden_tc(
      agg_item_h[0], agg_item_h[1], cnt_item, x_item,
      agg_user_h[0], agg_user_h[1], cnt_user, x_user,
      W1_l_u2i, W1_r_u2i, b1_u2i.reshape(1, F_HID),
      W1_l_i2u, W1_r_i2u, b1_i2u.reshape(1, F_HID),
      W2_l_u2i, W2_l_i2u, W2_r_u2i, b2_u2i.reshape(1, F_OUT),
      W2_r_i2u, b2_i2u.reshape(1, F_OUT))

  agg2_item, agg2_user = _seg_sum_l2(
      q_u, q_i, src_ui, dst_ui, src_iu, dst_iu, zeros_64)

  z_item, z_user = _final_tc(agg2_item, cnt_item, r2_item,
                             agg2_user, cnt_user, r2_user)

  el = edge_label_index.astype(jnp.int32)
  pad = LBL_PAD - E_LBL
  lshape = (NC * NS, LBL_CHUNKS_PER_TILE, LBL_CHUNK)
  el0_r = jnp.concatenate([el[0], jnp.zeros((pad,), jnp.int32)]).reshape(lshape)
  el1_r = jnp.concatenate([el[1], jnp.zeros((pad,), jnp.int32)]).reshape(lshape)

  pd = _pair_gather(z_user, z_item, el0_r, el1_r)
  pred = _pair_dot_tc(pd)
  return pred[:E_LBL]
